# Initial kernel scaffold; baseline (speedup 1.0000x reference)
import jax
import jax.numpy as jnp
from jax.experimental import pallas as pl


def kernel(x, adj_t):
    raise NotImplementedError("write your pallas kernel here")


# speedup vs baseline: 4.3829x; 4.3829x over previous
import functools

import jax
import jax.numpy as jnp
import numpy as np
from jax import lax
from jax.experimental import pallas as pl
from jax.experimental.pallas import tpu as pltpu
from jax.experimental.pallas import tpu_sc as plsc

_N = 10000
_D = 128
_E = 320000
_NS = 16
_NT = 32
_E_PER_TILE = _E // _NT
_CHUNK = 128
_NCHUNK = -(-_E_PER_TILE // _CHUNK)
_EPT_PAD = _NCHUNK * _CHUNK
_PAD = _EPT_PAD - _E_PER_TILE
_RCHUNK = 640
_RLAST = _N - 15 * _RCHUNK

_NOISE_SCALE = 1.0


def _init_term():
    n = jax.random.normal(
        jax.random.key(1234), (_N, _D), jnp.float32) * _NOISE_SCALE
    return jnp.stack([n, jnp.zeros((_N, _D), jnp.float32)], axis=0)


def _norm_body(x_ref, o_ref):
    x = x_ref[...]
    s = jnp.sum(x * x, axis=1, keepdims=True)
    o_ref[...] = x / jnp.maximum(jnp.sqrt(s), 1e-12)


def _combine_body(p_ref, o_ref):
    o_ref[...] = p_ref[0] + p_ref[1]


def _agg_body(xn_hbm, src_hbm, dst_hbm, init_hbm, out_hbm,
              sidx, didx, rows, acc, sem):
    cid = lax.axis_index("c")
    sid = lax.axis_index("s")
    row0 = pl.multiple_of(sid * _RCHUNK, 8)

    @pl.when(sid < 15)
    def _():
        pltpu.sync_copy(init_hbm.at[cid, pl.ds(row0, _RCHUNK)],
                        acc.at[pl.ds(row0, _RCHUNK)])

    @pl.when(sid == 15)
    def _():
        pltpu.sync_copy(init_hbm.at[cid, pl.ds(15 * _RCHUNK, _RLAST)],
                        acc.at[pl.ds(15 * _RCHUNK, _RLAST)])

    plsc.subcore_barrier()

    ebase = (cid * _NS + sid) * _EPT_PAD

    def body(i, carry):
        eb = pl.multiple_of(ebase + i * _CHUNK, _CHUNK)
        pltpu.sync_copy(src_hbm.at[pl.ds(eb, _CHUNK)], sidx)
        pltpu.sync_copy(dst_hbm.at[pl.ds(eb, _CHUNK)], didx)
        pltpu.async_copy(xn_hbm.at[sidx], rows, sem).wait()
        pltpu.sync_copy(rows, acc.at[didx], add=True)
        return carry

    lax.fori_loop(0, _NCHUNK, body, 0)

    plsc.subcore_barrier()

    @pl.when(sid < 15)
    def _():
        pltpu.sync_copy(acc.at[pl.ds(row0, _RCHUNK)],
                        out_hbm.at[cid, pl.ds(row0, _RCHUNK)])

    @pl.when(sid == 15)
    def _():
        pltpu.sync_copy(acc.at[pl.ds(15 * _RCHUNK, _RLAST)],
                        out_hbm.at[cid, pl.ds(15 * _RCHUNK, _RLAST)])


_DST_FILL = np.repeat(_N + (np.arange(_NT) % _NS), _PAD) \
    .reshape(_NT, _PAD).astype(np.int32)


def kernel(x, adj_t):
    adj = adj_t.astype(jnp.int32)
    srcp = jnp.pad(adj[0].reshape(_NT, _E_PER_TILE),
                   ((0, 0), (0, _PAD))).reshape(-1)
    dstp = jnp.concatenate(
        [adj[1].reshape(_NT, _E_PER_TILE), jnp.asarray(_DST_FILL)],
        axis=1).reshape(-1)

    xn = pl.pallas_call(
        _norm_body,
        out_shape=jax.ShapeDtypeStruct((_N, _D), jnp.float32),
    )(x)

    mesh = plsc.VectorSubcoreMesh(core_axis_name="c", subcore_axis_name="s")
    agg = functools.partial(
        pl.kernel,
        mesh=mesh,
        out_type=jax.ShapeDtypeStruct((2, _N, _D), jnp.float32),
        compiler_params=pltpu.CompilerParams(needs_layout_passes=False),
        scratch_types=[
            pltpu.VMEM((_CHUNK,), jnp.int32),
            pltpu.VMEM((_CHUNK,), jnp.int32),
            pltpu.VMEM((_CHUNK, _D), jnp.float32),
            pltpu.VMEM_SHARED((_N + _NS, _D), jnp.float32),
            pltpu.SemaphoreType.DMA,
        ],
    )(_agg_body)

    partials = agg(xn, srcp, dstp, _init_term())

    return pl.pallas_call(
        _combine_body,
        out_shape=jax.ShapeDtypeStruct((_N, _D), jnp.float32),
    )(partials)
